# Initial kernel scaffold; baseline (speedup 1.0000x reference)
import jax
import jax.numpy as jnp
from jax.experimental import pallas as pl


def kernel(x, k_aligned, v, M_prev, S_prev, poly_coeffs, alpha_W, alpha_b, eta_W, eta_b, theta_W, theta_b, gamma_W, gamma_b, out_W, out_b):
    raise NotImplementedError("write your pallas kernel here")


# speedup vs baseline: 1.2894x; 1.2894x over previous
import functools

import jax
import jax.numpy as jnp
from jax.experimental import pallas as pl
from jax.experimental.pallas import tpu as pltpu

W = 8
CHUNK = 8
NS_EPS = 1e-7


def _dot(a, b, dims):
    return jax.lax.dot_general(a, b, (dims, ((), ())),
                               preferred_element_type=jnp.float32)


def _atlas_kernel(x_ref, k_ref, v_ref, Mp_ref, Sp_ref, pc_ref,
                  aW_ref, ab_ref, eW_ref, eb_ref, tW_ref, tb_ref,
                  gW_ref, gb_ref, oW_ref, ob_ref,
                  out_ref, M_ref, S_ref,
                  bk_s, bv_s):
    c = pl.program_id(1)

    @pl.when(c == 0)
    def _init():
        M_ref[...] = Mp_ref[...]
        S_ref[...] = Sp_ref[...]
        bk_s[...] = jnp.zeros_like(bk_s)
        bv_s[...] = jnp.zeros_like(bv_s)

    x_c = x_ref[0]
    k_c = k_ref[0]
    v_c = v_ref[0]

    kphi = pc_ref[0:1, :] * k_c + pc_ref[1:2, :] * (k_c * k_c)

    aT = jax.nn.sigmoid(_dot(aW_ref[...], x_c, ((1,), (1,))) + ab_ref[...])
    eT = jax.nn.sigmoid(_dot(eW_ref[...], x_c, ((1,), (1,))) + eb_ref[...]) * 0.1
    tT = jax.nn.sigmoid(_dot(tW_ref[...], x_c, ((1,), (1,))) + tb_ref[...])
    g = jax.nn.sigmoid(_dot(x_c, gW_ref[...], ((1,), (0,))) + gb_ref[...])
    sg = jnp.sqrt(g)
    ks_c = sg * kphi
    vs_c = sg * v_c

    M = M_ref[0]
    S = S_ref[0]
    ys = []
    for j in range(CHUNK):
        bk_s[j:j + 1, :] = ks_c[j:j + 1, :]
        bv_s[j:j + 1, :] = vs_c[j:j + 1, :]
        Kw = bk_s[...]
        Vw = bv_s[...]
        pe = _dot(Kw, M, ((1,), (1,))) - Vw
        grad = _dot(pe, Kw, ((0,), (0,)))
        rc = jnp.where(c == 0, jnp.float32(1.0 / (j + 1)), jnp.float32(1.0 / W))
        S = tT[:, j:j + 1] * S + grad * rc
        nrm = jnp.sqrt(jnp.sum(S * S)) + NS_EPS
        X = S * (1.0 / nrm)
        XXt = _dot(X, X, ((1,), (1,)))
        So = 1.5 * X - 0.5 * _dot(XXt, X, ((1,), (0,)))
        M = aT[:, j:j + 1] * M - eT[:, j:j + 1] * So
        ys.append(_dot(kphi[j:j + 1, :], M, ((1,), (1,))))

    Y = jnp.concatenate(ys, axis=0)
    out_ref[0] = _dot(Y, oW_ref[...], ((1,), (1,))) + ob_ref[...]
    M_ref[0] = M
    S_ref[0] = S


@jax.jit
def kernel(x, k_aligned, v, M_prev, S_prev, poly_coeffs,
           alpha_W, alpha_b, eta_W, eta_b, theta_W, theta_b,
           gamma_W, gamma_b, out_W, out_b):
    B, L, D = x.shape
    nc = L // CHUNK

    row = lambda s: pl.BlockSpec(s, lambda b, c: (b, c, 0))
    bat = lambda: pl.BlockSpec((1, D, D), lambda b, c: (b, 0, 0))
    fix = lambda s: pl.BlockSpec(s, lambda b, c: (0,) * len(s))

    out, M_out, S_out = pl.pallas_call(
        _atlas_kernel,
        grid=(B, nc),
        in_specs=[
            row((1, CHUNK, D)),
            row((1, CHUNK, D)),
            row((1, CHUNK, D)),
            bat(),
            bat(),
            fix((2, D)),
            fix((D, D)), fix((D, 1)),
            fix((D, D)), fix((D, 1)),
            fix((D, D)), fix((D, 1)),
            fix((D, 1)), fix((1, 1)),
            fix((D, D)), fix((1, D)),
        ],
        out_specs=[
            row((1, CHUNK, D)),
            bat(),
            bat(),
        ],
        out_shape=[
            jax.ShapeDtypeStruct((B, L, D), jnp.float32),
            jax.ShapeDtypeStruct((B, D, D), jnp.float32),
            jax.ShapeDtypeStruct((B, D, D), jnp.float32),
        ],
        scratch_shapes=[
            pltpu.VMEM((W, D), jnp.float32),
            pltpu.VMEM((W, D), jnp.float32),
        ],
        compiler_params=pltpu.CompilerParams(
            dimension_semantics=("parallel", "arbitrary"),
        ),
        name="atlas_memory",
    )(x, k_aligned, v, M_prev, S_prev, poly_coeffs,
      alpha_W, alpha_b.reshape(D, 1), eta_W, eta_b.reshape(D, 1),
      theta_W, theta_b.reshape(D, 1), gamma_W.reshape(D, 1),
      gamma_b.reshape(1, 1), out_W, out_b.reshape(1, D))
    return (out, M_out, S_out)


# speedup vs baseline: 2.0954x; 1.6252x over previous
import functools

import jax
import jax.numpy as jnp
from jax.experimental import pallas as pl
from jax.experimental.pallas import tpu as pltpu

W = 8
CHUNK = 8
NS_EPS = 1e-7


def _dot(a, b, dims):
    return jax.lax.dot_general(a, b, (dims, ((), ())),
                               preferred_element_type=jnp.float32)


PAIR = 2


def _atlas_kernel(x_ref, k_ref, v_ref, Mp_ref, Sp_ref, pc_ref,
                  aW_ref, ab_ref, eW_ref, eb_ref, tW_ref, tb_ref,
                  gW_ref, gb_ref, oW_ref, ob_ref,
                  out_ref, M_ref, S_ref,
                  bk_s, bv_s):
    c = pl.program_id(1)

    @pl.when(c == 0)
    def _init():
        M_ref[...] = Mp_ref[...]
        S_ref[...] = Sp_ref[...]
        bk_s[...] = jnp.zeros_like(bk_s)
        bv_s[...] = jnp.zeros_like(bv_s)

    kphi_p, ks_p, vs_p, aT_p, eT_p, tT_p = [], [], [], [], [], []
    for i in range(PAIR):
        x_c = x_ref[i]
        k_c = k_ref[i]
        v_c = v_ref[i]
        kphi = pc_ref[0:1, :] * k_c + pc_ref[1:2, :] * (k_c * k_c)
        aT_p.append(jax.nn.sigmoid(_dot(aW_ref[...], x_c, ((1,), (1,))) + ab_ref[...]))
        eT_p.append(jax.nn.sigmoid(_dot(eW_ref[...], x_c, ((1,), (1,))) + eb_ref[...]) * 0.1)
        tT_p.append(jax.nn.sigmoid(_dot(tW_ref[...], x_c, ((1,), (1,))) + tb_ref[...]))
        g = jax.nn.sigmoid(_dot(x_c, gW_ref[...], ((1,), (0,))) + gb_ref[...])
        sg = jnp.sqrt(g)
        kphi_p.append(kphi)
        ks_p.append(sg * kphi)
        vs_p.append(sg * v_c)

    M_p = [M_ref[i] for i in range(PAIR)]
    S_p = [S_ref[i] for i in range(PAIR)]
    ys_p = [[] for _ in range(PAIR)]
    for j in range(CHUNK):
        rc = jnp.where(c == 0, jnp.float32(1.0 / (j + 1)), jnp.float32(1.0 / W))
        for i in range(PAIR):
            bk_s[i, j:j + 1, :] = ks_p[i][j:j + 1, :]
            bv_s[i, j:j + 1, :] = vs_p[i][j:j + 1, :]
            Kw = bk_s[i]
            Vw = bv_s[i]
            M, S = M_p[i], S_p[i]
            pe = (_dot(Kw, M, ((1,), (1,))) - Vw) * rc
            grad = _dot(pe, Kw, ((0,), (0,)))
            S = tT_p[i][:, j:j + 1] * S + grad
            nrm = jnp.sqrt(jnp.sum(S * S)) + NS_EPS
            SSt = _dot(S, S, ((1,), (1,)))
            SStS = _dot(SSt, S, ((1,), (0,)))
            ca = (1.5 / nrm) * eT_p[i][:, j:j + 1]
            cb = (0.5 / (nrm * nrm * nrm)) * eT_p[i][:, j:j + 1]
            M = aT_p[i][:, j:j + 1] * M - ca * S + cb * SStS
            M_p[i], S_p[i] = M, S
            ys_p[i].append(_dot(kphi_p[i][j:j + 1, :], M, ((1,), (1,))))

    for i in range(PAIR):
        Y = jnp.concatenate(ys_p[i], axis=0)
        out_ref[i] = _dot(Y, oW_ref[...], ((1,), (1,))) + ob_ref[...]
        M_ref[i] = M_p[i]
        S_ref[i] = S_p[i]


@jax.jit
def kernel(x, k_aligned, v, M_prev, S_prev, poly_coeffs,
           alpha_W, alpha_b, eta_W, eta_b, theta_W, theta_b,
           gamma_W, gamma_b, out_W, out_b):
    B, L, D = x.shape
    nc = L // CHUNK
    np_ = B // PAIR

    row = lambda: pl.BlockSpec((PAIR, CHUNK, D), lambda b, c: (b, c, 0))
    bat = lambda: pl.BlockSpec((PAIR, D, D), lambda b, c: (b, 0, 0))
    fix = lambda s: pl.BlockSpec(s, lambda b, c: (0,) * len(s))

    out, M_out, S_out = pl.pallas_call(
        _atlas_kernel,
        grid=(np_, nc),
        in_specs=[
            row(),
            row(),
            row(),
            bat(),
            bat(),
            fix((2, D)),
            fix((D, D)), fix((D, 1)),
            fix((D, D)), fix((D, 1)),
            fix((D, D)), fix((D, 1)),
            fix((D, 1)), fix((1, 1)),
            fix((D, D)), fix((1, D)),
        ],
        out_specs=[
            row(),
            bat(),
            bat(),
        ],
        out_shape=[
            jax.ShapeDtypeStruct((B, L, D), jnp.float32),
            jax.ShapeDtypeStruct((B, D, D), jnp.float32),
            jax.ShapeDtypeStruct((B, D, D), jnp.float32),
        ],
        scratch_shapes=[
            pltpu.VMEM((PAIR, W, D), jnp.float32),
            pltpu.VMEM((PAIR, W, D), jnp.float32),
        ],
        compiler_params=pltpu.CompilerParams(
            dimension_semantics=("parallel", "arbitrary"),
        ),
        name="atlas_memory",
    )(x, k_aligned, v, M_prev, S_prev, poly_coeffs,
      alpha_W, alpha_b.reshape(D, 1), eta_W, eta_b.reshape(D, 1),
      theta_W, theta_b.reshape(D, 1), gamma_W.reshape(D, 1),
      gamma_b.reshape(1, 1), out_W, out_b.reshape(1, D))
    return (out, M_out, S_out)


# speedup vs baseline: 2.1488x; 1.0255x over previous
import functools

import jax
import jax.numpy as jnp
from jax.experimental import pallas as pl
from jax.experimental.pallas import tpu as pltpu

W = 8
CHUNK = 8
NS_EPS = 1e-7


def _dot(a, b, dims):
    return jax.lax.dot_general(a, b, (dims, ((), ())),
                               preferred_element_type=jnp.float32)


PAIR = 4


def _atlas_kernel(x_ref, k_ref, v_ref, Mp_ref, Sp_ref, pc_ref,
                  aW_ref, ab_ref, eW_ref, eb_ref, tW_ref, tb_ref,
                  gW_ref, gb_ref, oW_ref, ob_ref,
                  out_ref, M_ref, S_ref,
                  bk_s, bv_s):
    c = pl.program_id(1)

    @pl.when(c == 0)
    def _init():
        M_ref[...] = Mp_ref[...]
        S_ref[...] = Sp_ref[...]
        bk_s[...] = jnp.zeros_like(bk_s)
        bv_s[...] = jnp.zeros_like(bv_s)

    kphi_p, ks_p, vs_p, aT_p, eT_p, tT_p = [], [], [], [], [], []
    for i in range(PAIR):
        x_c = x_ref[i]
        k_c = k_ref[i]
        v_c = v_ref[i]
        kphi = pc_ref[0:1, :] * k_c + pc_ref[1:2, :] * (k_c * k_c)
        aT_p.append(jax.nn.sigmoid(_dot(aW_ref[...], x_c, ((1,), (1,))) + ab_ref[...]))
        eT_p.append(jax.nn.sigmoid(_dot(eW_ref[...], x_c, ((1,), (1,))) + eb_ref[...]) * 0.1)
        tT_p.append(jax.nn.sigmoid(_dot(tW_ref[...], x_c, ((1,), (1,))) + tb_ref[...]))
        g = jax.nn.sigmoid(_dot(x_c, gW_ref[...], ((1,), (0,))) + gb_ref[...])
        sg = jnp.sqrt(g)
        kphi_p.append(kphi)
        ks_p.append(sg * kphi)
        vs_p.append(sg * v_c)

    M_p = [M_ref[i] for i in range(PAIR)]
    S_p = [S_ref[i] for i in range(PAIR)]
    ys_p = [[] for _ in range(PAIR)]
    for j in range(CHUNK):
        rc = jnp.where(c == 0, jnp.float32(1.0 / (j + 1)), jnp.float32(1.0 / W))
        for i in range(PAIR):
            bk_s[i, j:j + 1, :] = ks_p[i][j:j + 1, :]
            bv_s[i, j:j + 1, :] = vs_p[i][j:j + 1, :]
            Kw = bk_s[i]
            Vw = bv_s[i]
            M, S = M_p[i], S_p[i]
            pe = (_dot(Kw, M, ((1,), (1,))) - Vw) * rc
            grad = _dot(pe, Kw, ((0,), (0,)))
            S = tT_p[i][:, j:j + 1] * S + grad
            nrm = jnp.sqrt(jnp.sum(S * S)) + NS_EPS
            SSt = _dot(S, S, ((1,), (1,)))
            SStS = _dot(SSt, S, ((1,), (0,)))
            ca = (1.5 / nrm) * eT_p[i][:, j:j + 1]
            cb = (0.5 / (nrm * nrm * nrm)) * eT_p[i][:, j:j + 1]
            M = aT_p[i][:, j:j + 1] * M - ca * S + cb * SStS
            M_p[i], S_p[i] = M, S
            ys_p[i].append(_dot(kphi_p[i][j:j + 1, :], M, ((1,), (1,))))

    for i in range(PAIR):
        Y = jnp.concatenate(ys_p[i], axis=0)
        out_ref[i] = _dot(Y, oW_ref[...], ((1,), (1,))) + ob_ref[...]
        M_ref[i] = M_p[i]
        S_ref[i] = S_p[i]


@jax.jit
def kernel(x, k_aligned, v, M_prev, S_prev, poly_coeffs,
           alpha_W, alpha_b, eta_W, eta_b, theta_W, theta_b,
           gamma_W, gamma_b, out_W, out_b):
    B, L, D = x.shape
    nc = L // CHUNK
    np_ = B // PAIR

    row = lambda: pl.BlockSpec((PAIR, CHUNK, D), lambda b, c: (b, c, 0))
    bat = lambda: pl.BlockSpec((PAIR, D, D), lambda b, c: (b, 0, 0))
    fix = lambda s: pl.BlockSpec(s, lambda b, c: (0,) * len(s))

    out, M_out, S_out = pl.pallas_call(
        _atlas_kernel,
        grid=(np_, nc),
        in_specs=[
            row(),
            row(),
            row(),
            bat(),
            bat(),
            fix((2, D)),
            fix((D, D)), fix((D, 1)),
            fix((D, D)), fix((D, 1)),
            fix((D, D)), fix((D, 1)),
            fix((D, 1)), fix((1, 1)),
            fix((D, D)), fix((1, D)),
        ],
        out_specs=[
            row(),
            bat(),
            bat(),
        ],
        out_shape=[
            jax.ShapeDtypeStruct((B, L, D), jnp.float32),
            jax.ShapeDtypeStruct((B, D, D), jnp.float32),
            jax.ShapeDtypeStruct((B, D, D), jnp.float32),
        ],
        scratch_shapes=[
            pltpu.VMEM((PAIR, W, D), jnp.float32),
            pltpu.VMEM((PAIR, W, D), jnp.float32),
        ],
        compiler_params=pltpu.CompilerParams(
            dimension_semantics=("parallel", "arbitrary"),
        ),
        name="atlas_memory",
    )(x, k_aligned, v, M_prev, S_prev, poly_coeffs,
      alpha_W, alpha_b.reshape(D, 1), eta_W, eta_b.reshape(D, 1),
      theta_W, theta_b.reshape(D, 1), gamma_W.reshape(D, 1),
      gamma_b.reshape(1, 1), out_W, out_b.reshape(1, D))
    return (out, M_out, S_out)
